# baseline (speedup 1.0000x reference)
import functools

import jax
import jax.numpy as jnp
from jax import lax
from jax.experimental import pallas as pl
from jax.experimental.pallas import tpu as pltpu
from jax.experimental.pallas import tpu_sc as plsc

_N = 8192
_E = 131072
_LANES = 16
_NSUB = 16
_CHUNK = 128
_DK = 128


def _mm1(x, W1):
    n, k = x.shape
    h = W1.shape[1]
    h2 = h // 2
    bm = 1024

    def body(x_ref, w_ref, o0_ref, o1_ref):
        p = jnp.tanh(jnp.dot(x_ref[...], w_ref[...],
                             preferred_element_type=jnp.float32))
        o0_ref[...] = p[:, :h2]
        o1_ref[...] = p[:, h2:]

    return pl.pallas_call(
        body,
        grid=(n // bm,),
        in_specs=[
            pl.BlockSpec((bm, k), lambda i: (i, 0)),
            pl.BlockSpec((k, h), lambda i: (0, 0)),
        ],
        out_specs=[
            pl.BlockSpec((bm, h2), lambda i: (i, 0)),
            pl.BlockSpec((bm, h2), lambda i: (i, 0)),
        ],
        out_shape=[jax.ShapeDtypeStruct((n, h2), jnp.float32)] * 2,
    )(x, W1)


def _mm_pair(a0, a1, W, apply_tanh, concat_inputs):
    n = a0.shape[0]
    k0, k1 = a0.shape[1], a1.shape[1]
    h = W.shape[1]
    bm = 1024

    def body(a0_ref, a1_ref, w_ref, o_ref):
        if concat_inputs:
            p = jnp.dot(a0_ref[...], w_ref[:k0, :],
                        preferred_element_type=jnp.float32)
            p += jnp.dot(a1_ref[...], w_ref[k0:, :],
                         preferred_element_type=jnp.float32)
        else:
            p = jnp.dot(a0_ref[...] + a1_ref[...], w_ref[...],
                        preferred_element_type=jnp.float32)
        if apply_tanh:
            p = jnp.tanh(p)
        o_ref[...] = p

    return pl.pallas_call(
        body,
        grid=(n // bm,),
        in_specs=[
            pl.BlockSpec((bm, k0), lambda i: (i, 0)),
            pl.BlockSpec((bm, k1), lambda i: (i, 0)),
            pl.BlockSpec(W.shape, lambda i: (0, 0)),
        ],
        out_specs=pl.BlockSpec((bm, h), lambda i: (i, 0)),
        out_shape=jax.ShapeDtypeStruct((n, h), jnp.float32),
    )(a0, a1, W)


def _spmm_body(sup_ref, out_ref, acc, idx2, val_v, rows_v, sem_a, sem_b,
               ei, av, c, s, edge_base, edges_per_sub, dk_scale=_DK):
    rps = _N // _NSUB
    n_chunks = edges_per_sub // _CHUNK

    def zrow(r, carry):
        for j in range(_DK // _LANES):
            rows_v[0][r, pl.ds(j * _LANES, _LANES)] = jnp.zeros(
                (_LANES,), jnp.float32)
        return carry

    lax.fori_loop(0, _CHUNK, zrow, 0)
    for t in range(rps // _CHUNK):
        pltpu.sync_copy(rows_v[0], acc.at[pl.ds(s * rps + t * _CHUNK, _CHUNK)])
    plsc.subcore_barrier()

    def a_copies(g, b):
        base = edge_base + g * _CHUNK
        return (
            pltpu.make_async_copy(
                ei.at[0, pl.ds(base, _CHUNK)], idx2[b].at[0], sem_a[b]),
            pltpu.make_async_copy(
                ei.at[1, pl.ds(base, _CHUNK)], idx2[b].at[1], sem_a[b]),
            pltpu.make_async_copy(
                av.at[pl.ds(base, _CHUNK)], val_v[b], sem_a[b]),
        )

    def b_copy(b):
        return pltpu.make_async_copy(
            sup_ref.at[idx2[b].at[1]], rows_v[b % 2], sem_b[b])

    for d in a_copies(0, 0):
        d.start()
    for d in a_copies(1, 1):
        d.start()
    for d in a_copies(0, 0):
        d.wait()
    b_copy(0).start()

    def quad(i, carry):
        for k in range(4):
            g = 4 * i + k
            b_copy(k).wait()

            @pl.when(g + 2 < n_chunks)
            def _(k=k, g=g):
                for d in a_copies(g + 2, (k + 2) % 4):
                    d.start()

            @pl.when(g + 1 < n_chunks)
            def _(k=k, g=g):
                for d in a_copies(g + 1, (k + 1) % 4):
                    d.wait()
                b_copy((k + 1) % 4).start()

            @plsc.parallel_loop(0, _CHUNK, unroll=16)
            def _(e, k=k):
                v = plsc.load_gather(
                    val_v[k], [jnp.broadcast_to(e, (_LANES,))])
                for j in range(dk_scale // _LANES):
                    sl = pl.ds(j * _LANES, _LANES)
                    rows_v[k % 2][e, sl] = rows_v[k % 2][e, sl] * v

            pltpu.sync_copy(rows_v[k % 2], acc.at[idx2[k].at[0]], add=True)
        return carry

    lax.fori_loop(0, n_chunks // 4, quad, 0)
    plsc.subcore_barrier()
    rows = pl.ds(s * rps, rps)
    pltpu.sync_copy(acc.at[rows], out_ref.at[rows])


_SPMM_SCRATCH = [
    [pltpu.VMEM((2, _CHUNK), jnp.int32)] * 4,
    [pltpu.VMEM((_CHUNK,), jnp.float32)] * 4,
    [pltpu.VMEM((_CHUNK, _DK), jnp.float32)] * 2,
    pltpu.VMEM_SHARED((_N, _DK), jnp.float32),
    [pltpu.SemaphoreType.DMA] * 4,
    [pltpu.SemaphoreType.DMA] * 4,
]
_SC_PARAMS = pltpu.CompilerParams(needs_layout_passes=False)
_SC_MESH = plsc.VectorSubcoreMesh(core_axis_name="c", subcore_axis_name="s")


@functools.partial(
    pl.kernel,
    out_type=[jax.ShapeDtypeStruct((_N, _DK), jnp.float32)] * 2,
    mesh=_SC_MESH,
    scratch_types=_SPMM_SCRATCH,
    compiler_params=_SC_PARAMS,
)
def _spmm_featsplit(sup0, sup1, ei, av, out0, out1,
                    idx2, val_v, rows_v, acc, sem_a, sem_b):
    c = lax.axis_index("c")
    s = lax.axis_index("s")
    ep = _E // _NSUB

    @pl.when(c == 0)
    def _():
        _spmm_body(sup0, out0, acc, idx2, val_v, rows_v, sem_a, sem_b,
                   ei, av, c, s, s * ep, ep)

    @pl.when(c == 1)
    def _():
        _spmm_body(sup1, out1, acc, idx2, val_v, rows_v, sem_a, sem_b,
                   ei, av, c, s, s * ep, ep)


@functools.lru_cache(maxsize=None)
def _make_spmm_edgesplit(dk_scale):
    @functools.partial(
        pl.kernel,
        out_type=[jax.ShapeDtypeStruct((_N, _DK), jnp.float32)] * 2,
        mesh=_SC_MESH,
        scratch_types=_SPMM_SCRATCH,
        compiler_params=_SC_PARAMS,
    )
    def _spmm_edgesplit(sup, ei, av, out0, out1,
                        idx2, val_v, rows_v, acc, sem_a, sem_b):
        c = lax.axis_index("c")
        s = lax.axis_index("s")
        ep = _E // 2 // _NSUB

        @pl.when(c == 0)
        def _():
            _spmm_body(sup, out0, acc, idx2, val_v, rows_v, sem_a, sem_b,
                       ei, av, c, s, s * ep, ep, dk_scale)

        @pl.when(c == 1)
        def _():
            _spmm_body(sup, out1, acc, idx2, val_v, rows_v, sem_a, sem_b,
                       ei, av, c, s, _E // 2 + s * ep, ep, dk_scale)

    return _spmm_edgesplit


def _zadj(q0, q1, l):
    n, k = q0.shape
    bm, bn = 1024, 4096

    def body(a0_ref, a1_ref, b0_ref, b1_ref, zx_ref, o_ref):
        zi = (a0_ref[...] + a1_ref[...])[:, :l]
        zj = (b0_ref[...] + b1_ref[...])[:, :l]
        zx_ref[...] = zi
        p = jax.lax.dot_general(zi, zj, (((1,), (1,)), ((), ())),
                                preferred_element_type=jnp.float32)
        o_ref[...] = 1.0 / (1.0 + jnp.exp(-p))

    return pl.pallas_call(
        body,
        grid=(n // bm, n // bn),
        in_specs=[
            pl.BlockSpec((bm, k), lambda i, j: (i, 0)),
            pl.BlockSpec((bm, k), lambda i, j: (i, 0)),
            pl.BlockSpec((bn, k), lambda i, j: (j, 0)),
            pl.BlockSpec((bn, k), lambda i, j: (j, 0)),
        ],
        out_specs=[
            pl.BlockSpec((bm, l), lambda i, j: (i, 0)),
            pl.BlockSpec((bm, bn), lambda i, j: (i, j)),
        ],
        out_shape=[
            jax.ShapeDtypeStruct((n, l), jnp.float32),
            jax.ShapeDtypeStruct((n, n), jnp.float32),
        ],
    )(q0, q1, q0, q1)


def kernel(x, edge_index, adj_vals, W1, W2, W3):
    l = W3.shape[1]
    W3p = jnp.pad(W3, ((0, 0), (0, _DK - l)))
    s0, s1 = _mm1(x, W1)
    h0, h1 = _spmm_featsplit(s0, s1, edge_index, adj_vals)
    sup2 = _mm_pair(h0, h1, W2, True, concat_inputs=True)
    p0, p1 = _make_spmm_edgesplit(_DK)(sup2, edge_index, adj_vals)
    sup3 = _mm_pair(p0, p1, W3p, False, concat_inputs=False)
    q0, q1 = _make_spmm_edgesplit(l)(sup3, edge_index, adj_vals)
    z_x, z_adj = _zadj(q0, q1, l)
    return (z_x, z_adj)


# speedup vs baseline: 1.0096x; 1.0096x over previous
import functools

import jax
import jax.numpy as jnp
from jax import lax
from jax.experimental import pallas as pl
from jax.experimental.pallas import tpu as pltpu
from jax.experimental.pallas import tpu_sc as plsc

_N = 8192
_E = 131072
_LANES = 16
_NSUB = 16
_CHUNK = 128
_DK = 128


def _mm1(x, W1):
    n, k = x.shape
    h = W1.shape[1]
    h2 = h // 2
    bm = 1024

    def body(x_ref, w_ref, o0_ref, o1_ref):
        p = jnp.tanh(jnp.dot(x_ref[...], w_ref[...],
                             preferred_element_type=jnp.float32))
        o0_ref[...] = p[:, :h2]
        o1_ref[...] = p[:, h2:]

    return pl.pallas_call(
        body,
        grid=(n // bm,),
        in_specs=[
            pl.BlockSpec((bm, k), lambda i: (i, 0)),
            pl.BlockSpec((k, h), lambda i: (0, 0)),
        ],
        out_specs=[
            pl.BlockSpec((bm, h2), lambda i: (i, 0)),
            pl.BlockSpec((bm, h2), lambda i: (i, 0)),
        ],
        out_shape=[jax.ShapeDtypeStruct((n, h2), jnp.float32)] * 2,
    )(x, W1)


def _mm_pair(a0, a1, W, apply_tanh, concat_inputs):
    n = a0.shape[0]
    k0, k1 = a0.shape[1], a1.shape[1]
    h = W.shape[1]
    bm = 1024

    def body(a0_ref, a1_ref, w_ref, o_ref):
        if concat_inputs:
            p = jnp.dot(a0_ref[...], w_ref[:k0, :],
                        preferred_element_type=jnp.float32)
            p += jnp.dot(a1_ref[...], w_ref[k0:, :],
                         preferred_element_type=jnp.float32)
        else:
            p = jnp.dot(a0_ref[...] + a1_ref[...], w_ref[...],
                        preferred_element_type=jnp.float32)
        if apply_tanh:
            p = jnp.tanh(p)
        o_ref[...] = p

    return pl.pallas_call(
        body,
        grid=(n // bm,),
        in_specs=[
            pl.BlockSpec((bm, k0), lambda i: (i, 0)),
            pl.BlockSpec((bm, k1), lambda i: (i, 0)),
            pl.BlockSpec(W.shape, lambda i: (0, 0)),
        ],
        out_specs=pl.BlockSpec((bm, h), lambda i: (i, 0)),
        out_shape=jax.ShapeDtypeStruct((n, h), jnp.float32),
    )(a0, a1, W)


def _spmm_body(sup_ref, out_ref, acc, idx2, val_v, rows_v, sem_a, sem_b,
               ei, av, c, s, edge_base, edges_per_sub, dk_scale=_DK):
    rps = _N // _NSUB
    n_chunks = edges_per_sub // _CHUNK

    def zrow(r, carry):
        for j in range(_DK // _LANES):
            rows_v[0][r, pl.ds(j * _LANES, _LANES)] = jnp.zeros(
                (_LANES,), jnp.float32)
        return carry

    lax.fori_loop(0, _CHUNK, zrow, 0)
    for t in range(rps // _CHUNK):
        pltpu.sync_copy(rows_v[0], acc.at[pl.ds(s * rps + t * _CHUNK, _CHUNK)])
    plsc.subcore_barrier()

    def a_copies(g, b):
        base = edge_base + g * _CHUNK
        return (
            pltpu.make_async_copy(
                ei.at[0, pl.ds(base, _CHUNK)], idx2[b].at[0], sem_a[b]),
            pltpu.make_async_copy(
                ei.at[1, pl.ds(base, _CHUNK)], idx2[b].at[1], sem_a[b]),
            pltpu.make_async_copy(
                av.at[pl.ds(base, _CHUNK)], val_v[b], sem_a[b]),
        )

    def b_copy(b):
        return pltpu.make_async_copy(
            sup_ref.at[idx2[b].at[1]], rows_v[b % 2], sem_b[b])

    for d in a_copies(0, 0):
        d.start()
    for d in a_copies(1, 1):
        d.start()
    for d in a_copies(0, 0):
        d.wait()
    b_copy(0).start()

    def quad(i, carry):
        for k in range(4):
            g = 4 * i + k
            b_copy(k).wait()

            @pl.when(g + 2 < n_chunks)
            def _(k=k, g=g):
                for d in a_copies(g + 2, (k + 2) % 4):
                    d.start()

            @pl.when(g + 1 < n_chunks)
            def _(k=k, g=g):
                for d in a_copies(g + 1, (k + 1) % 4):
                    d.wait()
                b_copy((k + 1) % 4).start()

            @plsc.parallel_loop(0, _CHUNK, unroll=16)
            def _(e, k=k):
                v = plsc.load_gather(
                    val_v[k], [jnp.broadcast_to(e, (_LANES,))])
                for j in range(dk_scale // _LANES):
                    sl = pl.ds(j * _LANES, _LANES)
                    rows_v[k % 2][e, sl] = rows_v[k % 2][e, sl] * v

            pltpu.sync_copy(rows_v[k % 2], acc.at[idx2[k].at[0]], add=True)
        return carry

    lax.fori_loop(0, n_chunks // 4, quad, 0)
    plsc.subcore_barrier()
    rows = pl.ds(s * rps, rps)
    pltpu.sync_copy(acc.at[rows], out_ref.at[rows])


_SPMM_SCRATCH = [
    [pltpu.VMEM((2, _CHUNK), jnp.int32)] * 4,
    [pltpu.VMEM((_CHUNK,), jnp.float32)] * 4,
    [pltpu.VMEM((_CHUNK, _DK), jnp.float32)] * 2,
    pltpu.VMEM_SHARED((_N, _DK), jnp.float32),
    [pltpu.SemaphoreType.DMA] * 4,
    [pltpu.SemaphoreType.DMA] * 4,
]
_SC_PARAMS = pltpu.CompilerParams(needs_layout_passes=False)
_SC_MESH = plsc.VectorSubcoreMesh(core_axis_name="c", subcore_axis_name="s")


@functools.partial(
    pl.kernel,
    out_type=[jax.ShapeDtypeStruct((_N, _DK), jnp.float32)] * 2,
    mesh=_SC_MESH,
    scratch_types=_SPMM_SCRATCH,
    compiler_params=_SC_PARAMS,
)
def _spmm_featsplit(sup0, sup1, ei, av, out0, out1,
                    idx2, val_v, rows_v, acc, sem_a, sem_b):
    c = lax.axis_index("c")
    s = lax.axis_index("s")
    ep = _E // _NSUB

    @pl.when(c == 0)
    def _():
        _spmm_body(sup0, out0, acc, idx2, val_v, rows_v, sem_a, sem_b,
                   ei, av, c, s, s * ep, ep)

    @pl.when(c == 1)
    def _():
        _spmm_body(sup1, out1, acc, idx2, val_v, rows_v, sem_a, sem_b,
                   ei, av, c, s, s * ep, ep)


@functools.lru_cache(maxsize=None)
def _make_spmm_edgesplit(dk_scale):
    @functools.partial(
        pl.kernel,
        out_type=[jax.ShapeDtypeStruct((_N, _DK), jnp.float32)] * 2,
        mesh=_SC_MESH,
        scratch_types=_SPMM_SCRATCH,
        compiler_params=_SC_PARAMS,
    )
    def _spmm_edgesplit(sup, ei, av, out0, out1,
                        idx2, val_v, rows_v, acc, sem_a, sem_b):
        c = lax.axis_index("c")
        s = lax.axis_index("s")
        ep = _E // 2 // _NSUB

        @pl.when(c == 0)
        def _():
            _spmm_body(sup, out0, acc, idx2, val_v, rows_v, sem_a, sem_b,
                       ei, av, c, s, s * ep, ep, dk_scale)

        @pl.when(c == 1)
        def _():
            _spmm_body(sup, out1, acc, idx2, val_v, rows_v, sem_a, sem_b,
                       ei, av, c, s, _E // 2 + s * ep, ep, dk_scale)

    return _spmm_edgesplit


def _zx_combine(a0, a1, l):
    n, k = a0.shape
    bm = 1024

    def body(a0_ref, a1_ref, o_ref):
        o_ref[...] = (a0_ref[...] + a1_ref[...])[:, :l]

    return pl.pallas_call(
        body,
        grid=(n // bm,),
        in_specs=[
            pl.BlockSpec((bm, k), lambda i: (i, 0)),
            pl.BlockSpec((bm, k), lambda i: (i, 0)),
        ],
        out_specs=pl.BlockSpec((bm, l), lambda i: (i, 0)),
        out_shape=jax.ShapeDtypeStruct((n, l), jnp.float32),
    )(a0, a1)


def _zadj(zx, zxt):
    n, l = zx.shape
    bm, bn = 1024, 4096

    def body(a_ref, b_ref, o_ref):
        p = jnp.dot(a_ref[...], b_ref[...], preferred_element_type=jnp.float32)
        o_ref[...] = 1.0 / (1.0 + jnp.exp(-p))

    return pl.pallas_call(
        body,
        grid=(n // bm, n // bn),
        in_specs=[
            pl.BlockSpec((bm, l), lambda i, j: (i, 0)),
            pl.BlockSpec((l, bn), lambda i, j: (0, j)),
        ],
        out_specs=pl.BlockSpec((bm, bn), lambda i, j: (i, j)),
        out_shape=jax.ShapeDtypeStruct((n, n), jnp.float32),
    )(zx, zxt)


def kernel(x, edge_index, adj_vals, W1, W2, W3):
    l = W3.shape[1]
    W3p = jnp.pad(W3, ((0, 0), (0, _DK - l)))
    s0, s1 = _mm1(x, W1)
    h0, h1 = _spmm_featsplit(s0, s1, edge_index, adj_vals)
    sup2 = _mm_pair(h0, h1, W2, True, concat_inputs=True)
    p0, p1 = _make_spmm_edgesplit(_DK)(sup2, edge_index, adj_vals)
    sup3 = _mm_pair(p0, p1, W3p, False, concat_inputs=False)
    q0, q1 = _make_spmm_edgesplit(l)(sup3, edge_index, adj_vals)
    z_x = _zx_combine(q0, q1, l)
    z_adj = _zadj(z_x, z_x.T)
    return (z_x, z_adj)


# speedup vs baseline: 1.0301x; 1.0203x over previous
import functools

import jax
import jax.numpy as jnp
from jax import lax
from jax.experimental import pallas as pl
from jax.experimental.pallas import tpu as pltpu
from jax.experimental.pallas import tpu_sc as plsc

_N = 8192
_E = 131072
_LANES = 16
_NSUB = 16
_CHUNK = 128
_DK = 128


def _mm1(x, W1):
    n, k = x.shape
    h = W1.shape[1]
    h2 = h // 2
    bm = 1024

    def body(x_ref, w_ref, o0_ref, o1_ref):
        p = jnp.tanh(jnp.dot(x_ref[...], w_ref[...],
                             preferred_element_type=jnp.float32))
        o0_ref[...] = p[:, :h2]
        o1_ref[...] = p[:, h2:]

    return pl.pallas_call(
        body,
        grid=(n // bm,),
        in_specs=[
            pl.BlockSpec((bm, k), lambda i: (i, 0)),
            pl.BlockSpec((k, h), lambda i: (0, 0)),
        ],
        out_specs=[
            pl.BlockSpec((bm, h2), lambda i: (i, 0)),
            pl.BlockSpec((bm, h2), lambda i: (i, 0)),
        ],
        out_shape=[jax.ShapeDtypeStruct((n, h2), jnp.float32)] * 2,
    )(x, W1)


def _mm_pair(a0, a1, W, apply_tanh, concat_inputs):
    n = a0.shape[0]
    k0, k1 = a0.shape[1], a1.shape[1]
    h = W.shape[1]
    bm = 1024

    def body(a0_ref, a1_ref, w_ref, o_ref):
        if concat_inputs:
            p = jnp.dot(a0_ref[...], w_ref[:k0, :],
                        preferred_element_type=jnp.float32)
            p += jnp.dot(a1_ref[...], w_ref[k0:, :],
                         preferred_element_type=jnp.float32)
        else:
            p = jnp.dot(a0_ref[...] + a1_ref[...], w_ref[...],
                        preferred_element_type=jnp.float32)
        if apply_tanh:
            p = jnp.tanh(p)
        o_ref[...] = p

    return pl.pallas_call(
        body,
        grid=(n // bm,),
        in_specs=[
            pl.BlockSpec((bm, k0), lambda i: (i, 0)),
            pl.BlockSpec((bm, k1), lambda i: (i, 0)),
            pl.BlockSpec(W.shape, lambda i: (0, 0)),
        ],
        out_specs=pl.BlockSpec((bm, h), lambda i: (i, 0)),
        out_shape=jax.ShapeDtypeStruct((n, h), jnp.float32),
    )(a0, a1, W)


def _spmm_body(sup_ref, out_ref, acc, idx2, val_v, rows_v, sem_a, sem_b,
               ei, av, c, s, edge_base, edges_per_sub, dk_scale=_DK):
    rps = _N // _NSUB
    n_chunks = edges_per_sub // _CHUNK

    def zrow(r, carry):
        for j in range(_DK // _LANES):
            rows_v[0][r, pl.ds(j * _LANES, _LANES)] = jnp.zeros(
                (_LANES,), jnp.float32)
        return carry

    lax.fori_loop(0, _CHUNK, zrow, 0)
    for t in range(rps // _CHUNK):
        pltpu.sync_copy(rows_v[0], acc.at[pl.ds(s * rps + t * _CHUNK, _CHUNK)])
    plsc.subcore_barrier()

    def a_copies(g, b):
        base = edge_base + g * _CHUNK
        return (
            pltpu.make_async_copy(
                ei.at[0, pl.ds(base, _CHUNK)], idx2[b].at[0], sem_a[b]),
            pltpu.make_async_copy(
                ei.at[1, pl.ds(base, _CHUNK)], idx2[b].at[1], sem_a[b]),
            pltpu.make_async_copy(
                av.at[pl.ds(base, _CHUNK)], val_v[b], sem_a[b]),
        )

    def b_copy(b):
        return pltpu.make_async_copy(
            sup_ref.at[idx2[b].at[1]], rows_v[b % 2], sem_b[b])

    for d in a_copies(0, 0):
        d.start()
    for d in a_copies(1, 1):
        d.start()
    for d in a_copies(0, 0):
        d.wait()
    b_copy(0).start()

    def quad(i, carry):
        for k in range(4):
            g = 4 * i + k
            b_copy(k).wait()

            @pl.when(g + 2 < n_chunks)
            def _(k=k, g=g):
                for d in a_copies(g + 2, (k + 2) % 4):
                    d.start()

            @pl.when(g + 1 < n_chunks)
            def _(k=k, g=g):
                for d in a_copies(g + 1, (k + 1) % 4):
                    d.wait()
                b_copy((k + 1) % 4).start()

            @plsc.parallel_loop(0, _CHUNK, unroll=8)
            def _(e, k=k):
                v = plsc.load_gather(
                    val_v[k], [jnp.broadcast_to(e, (_LANES,))])
                for j in range(dk_scale // _LANES):
                    sl = pl.ds(j * _LANES, _LANES)
                    rows_v[k % 2][e, sl] = rows_v[k % 2][e, sl] * v

            pltpu.sync_copy(rows_v[k % 2], acc.at[idx2[k].at[0]], add=True)
        return carry

    lax.fori_loop(0, n_chunks // 4, quad, 0)
    plsc.subcore_barrier()
    rows = pl.ds(s * rps, rps)
    pltpu.sync_copy(acc.at[rows], out_ref.at[rows])


_SPMM_SCRATCH = [
    [pltpu.VMEM((2, _CHUNK), jnp.int32)] * 4,
    [pltpu.VMEM((_CHUNK,), jnp.float32)] * 4,
    [pltpu.VMEM((_CHUNK, _DK), jnp.float32)] * 2,
    pltpu.VMEM_SHARED((_N, _DK), jnp.float32),
    [pltpu.SemaphoreType.DMA] * 4,
    [pltpu.SemaphoreType.DMA] * 4,
]
_SC_PARAMS = pltpu.CompilerParams(needs_layout_passes=False)
_SC_MESH = plsc.VectorSubcoreMesh(core_axis_name="c", subcore_axis_name="s")


@functools.partial(
    pl.kernel,
    out_type=[jax.ShapeDtypeStruct((_N, _DK), jnp.float32)] * 2,
    mesh=_SC_MESH,
    scratch_types=_SPMM_SCRATCH,
    compiler_params=_SC_PARAMS,
)
def _spmm_featsplit(sup0, sup1, ei, av, out0, out1,
                    idx2, val_v, rows_v, acc, sem_a, sem_b):
    c = lax.axis_index("c")
    s = lax.axis_index("s")
    ep = _E // _NSUB

    @pl.when(c == 0)
    def _():
        _spmm_body(sup0, out0, acc, idx2, val_v, rows_v, sem_a, sem_b,
                   ei, av, c, s, s * ep, ep)

    @pl.when(c == 1)
    def _():
        _spmm_body(sup1, out1, acc, idx2, val_v, rows_v, sem_a, sem_b,
                   ei, av, c, s, s * ep, ep)


@functools.lru_cache(maxsize=None)
def _make_spmm_edgesplit(dk_scale):
    @functools.partial(
        pl.kernel,
        out_type=[jax.ShapeDtypeStruct((_N, _DK), jnp.float32)] * 2,
        mesh=_SC_MESH,
        scratch_types=_SPMM_SCRATCH,
        compiler_params=_SC_PARAMS,
    )
    def _spmm_edgesplit(sup, ei, av, out0, out1,
                        idx2, val_v, rows_v, acc, sem_a, sem_b):
        c = lax.axis_index("c")
        s = lax.axis_index("s")
        ep = _E // 2 // _NSUB

        @pl.when(c == 0)
        def _():
            _spmm_body(sup, out0, acc, idx2, val_v, rows_v, sem_a, sem_b,
                       ei, av, c, s, s * ep, ep, dk_scale)

        @pl.when(c == 1)
        def _():
            _spmm_body(sup, out1, acc, idx2, val_v, rows_v, sem_a, sem_b,
                       ei, av, c, s, _E // 2 + s * ep, ep, dk_scale)

    return _spmm_edgesplit


def _zx_combine(a0, a1, l):
    n, k = a0.shape
    bm = 1024

    def body(a0_ref, a1_ref, o_ref):
        o_ref[...] = (a0_ref[...] + a1_ref[...])[:, :l]

    return pl.pallas_call(
        body,
        grid=(n // bm,),
        in_specs=[
            pl.BlockSpec((bm, k), lambda i: (i, 0)),
            pl.BlockSpec((bm, k), lambda i: (i, 0)),
        ],
        out_specs=pl.BlockSpec((bm, l), lambda i: (i, 0)),
        out_shape=jax.ShapeDtypeStruct((n, l), jnp.float32),
    )(a0, a1)


def _zadj(zx, zxt):
    n, l = zx.shape
    bm, bn = 1024, 4096

    def body(a_ref, b_ref, o_ref):
        p = jnp.dot(a_ref[...], b_ref[...], preferred_element_type=jnp.float32)
        o_ref[...] = 1.0 / (1.0 + jnp.exp(-p))

    return pl.pallas_call(
        body,
        grid=(n // bm, n // bn),
        in_specs=[
            pl.BlockSpec((bm, l), lambda i, j: (i, 0)),
            pl.BlockSpec((l, bn), lambda i, j: (0, j)),
        ],
        out_specs=pl.BlockSpec((bm, bn), lambda i, j: (i, j)),
        out_shape=jax.ShapeDtypeStruct((n, n), jnp.float32),
    )(zx, zxt)


def kernel(x, edge_index, adj_vals, W1, W2, W3):
    l = W3.shape[1]
    W3p = jnp.pad(W3, ((0, 0), (0, _DK - l)))
    s0, s1 = _mm1(x, W1)
    h0, h1 = _spmm_featsplit(s0, s1, edge_index, adj_vals)
    sup2 = _mm_pair(h0, h1, W2, True, concat_inputs=True)
    p0, p1 = _make_spmm_edgesplit(_DK)(sup2, edge_index, adj_vals)
    sup3 = _mm_pair(p0, p1, W3p, False, concat_inputs=False)
    q0, q1 = _make_spmm_edgesplit(l)(sup3, edge_index, adj_vals)
    z_x = _zx_combine(q0, q1, l)
    z_adj = _zadj(z_x, z_x.T)
    return (z_x, z_adj)
